# Initial kernel scaffold; baseline (speedup 1.0000x reference)
import jax
import jax.numpy as jnp
from jax.experimental import pallas as pl


def kernel(x, edge_index, batch, W1, b1, W2, b2, Wl1, bl1, Wl2, bl2):
    raise NotImplementedError("write your pallas kernel here")


# speedup vs baseline: 10.1069x; 10.1069x over previous
import functools

import jax
import jax.numpy as jnp
from jax import lax
from jax.experimental import pallas as pl
from jax.experimental.pallas import tpu as pltpu
from jax.experimental.pallas import tpu_sc as plsc

N = 10000
E = 320000
D = 128
NG = 16

NC = 2
NS = 16
NW = NC * NS

NPAD = 10240
ROWS_PT = NPAD // NS

B = 128
EPW = ((E // NW + B - 1) // B) * B
EPAD = EPW * NW
NCHUNK = EPW // B
CW = 128

_mesh = plsc.VectorSubcoreMesh(core_axis_name="c", subcore_axis_name="s")


@functools.partial(
    pl.kernel,
    out_type=jax.ShapeDtypeStruct((NC, NPAD, CW), jnp.float32),
    mesh=_mesh,
    scratch_types=[
        pltpu.VMEM((B,), jnp.int32),
        pltpu.VMEM((B, CW), jnp.float32),
        pltpu.VMEM_SHARED((NPAD, CW), jnp.float32),
        pltpu.SemaphoreType.DMA,
    ],
)
def _deg_kernel(col_hbm, ones_hbm, zeros_hbm, out_hbm, col_v, ones_v, cnt_sh, sem):
    c = lax.axis_index("c")
    s = lax.axis_index("s")
    wid = c * NS + s
    r0 = s * ROWS_PT
    pltpu.sync_copy(zeros_hbm.at[pl.ds(r0, ROWS_PT)], cnt_sh.at[pl.ds(r0, ROWS_PT)])
    pltpu.sync_copy(ones_hbm, ones_v)
    plsc.subcore_barrier()

    def body(j, carry):
        base = pl.multiple_of(wid * EPW + j * B, B)
        pltpu.sync_copy(col_hbm.at[pl.ds(base, B)], col_v)
        pltpu.sync_copy(ones_v, cnt_sh.at[col_v], add=True)
        return carry

    lax.fori_loop(0, NCHUNK, body, 0)
    plsc.subcore_barrier()
    pltpu.sync_copy(cnt_sh.at[pl.ds(r0, ROWS_PT)], out_hbm.at[c, pl.ds(r0, ROWS_PT)])


@functools.partial(
    pl.kernel,
    out_type=jax.ShapeDtypeStruct((NC, NPAD, D), jnp.float32),
    mesh=_mesh,
    scratch_types=[
        pltpu.VMEM((B,), jnp.int32),
        pltpu.VMEM((B,), jnp.int32),
        pltpu.VMEM((B, D), jnp.float32),
        pltpu.VMEM_SHARED((NPAD, D), jnp.float32),
        pltpu.SemaphoreType.DMA,
    ],
)
def _agg_kernel(y_hbm, row_hbm, col_hbm, zeros_hbm, out_hbm,
                row_v, col_v, rows_v, acc_sh, sem):
    c = lax.axis_index("c")
    s = lax.axis_index("s")
    wid = c * NS + s
    r0 = s * ROWS_PT
    pltpu.sync_copy(zeros_hbm.at[pl.ds(r0, ROWS_PT)], acc_sh.at[pl.ds(r0, ROWS_PT)])
    plsc.subcore_barrier()

    def body(j, carry):
        base = pl.multiple_of(wid * EPW + j * B, B)
        pltpu.sync_copy(row_hbm.at[pl.ds(base, B)], row_v)
        pltpu.sync_copy(col_hbm.at[pl.ds(base, B)], col_v)
        pltpu.async_copy(y_hbm.at[row_v], rows_v, sem).wait()
        pltpu.sync_copy(rows_v, acc_sh.at[col_v], add=True)
        return carry

    lax.fori_loop(0, NCHUNK, body, 0)
    plsc.subcore_barrier()
    pltpu.sync_copy(acc_sh.at[pl.ds(r0, ROWS_PT)], out_hbm.at[c, pl.ds(r0, ROWS_PT)])


def _dis_from_cnt(cnt_t_ref):
    deg = cnt_t_ref[:, 0:1] + cnt_t_ref[:, 1:2] + 1.0
    return lax.rsqrt(deg)


def _tc_pre_body(cnt_t_ref, x_ref, w1_ref, y_ref, xw_ref):
    dis = _dis_from_cnt(cnt_t_ref)
    xw = jnp.dot(x_ref[...], w1_ref[...], preferred_element_type=jnp.float32)
    xw_ref[...] = xw
    y_ref[...] = xw * dis


_tc_pre = pl.pallas_call(
    _tc_pre_body,
    out_shape=[
        jax.ShapeDtypeStruct((NPAD, D), jnp.float32),
        jax.ShapeDtypeStruct((NPAD, D), jnp.float32),
    ],
)


def _tc_mid_body(acc_ref, cnt_t_ref, xw_ref, b1_ref, w2_ref, y2_ref, xw2_ref):
    dis = _dis_from_cnt(cnt_t_ref)
    agg = acc_ref[0] + acc_ref[1]
    h = jax.nn.relu(dis * agg + (dis * dis) * xw_ref[...] + b1_ref[...])
    xw2 = jnp.dot(h, w2_ref[...], preferred_element_type=jnp.float32)
    xw2_ref[...] = xw2
    y2_ref[...] = xw2 * dis


_tc_mid = pl.pallas_call(
    _tc_mid_body,
    out_shape=[
        jax.ShapeDtypeStruct((NPAD, D), jnp.float32),
        jax.ShapeDtypeStruct((NPAD, D), jnp.float32),
    ],
)


def _tc_fin_body(acc_ref, cnt_t_ref, xw_ref, b2_ref, batch_ref,
                 wl1_ref, bl1_ref, wl2_ref, bl2_ref, out_ref):
    dis = _dis_from_cnt(cnt_t_ref)
    agg = acc_ref[0] + acc_ref[1]
    h = jax.nn.relu(dis * agg + (dis * dis) * xw_ref[...] + b2_ref[...])
    gidx = lax.broadcasted_iota(jnp.int32, (NG, NPAD), 0)
    mask = (jnp.broadcast_to(batch_ref[...], (NG, NPAD)) == gidx).astype(jnp.float32)
    sums = jnp.dot(mask, h, preferred_element_type=jnp.float32)
    cnt = jnp.maximum(jnp.sum(mask, axis=1, keepdims=True), 1.0)
    g = sums / cnt
    g = jax.nn.relu(jnp.dot(g, wl1_ref[...], preferred_element_type=jnp.float32)
                    + bl1_ref[...])
    out_ref[...] = (jnp.dot(g, wl2_ref[...], preferred_element_type=jnp.float32)
                    + bl2_ref[...])


_tc_fin = pl.pallas_call(
    _tc_fin_body,
    out_shape=jax.ShapeDtypeStruct((NG, 16), jnp.float32),
)


def kernel(x, edge_index, batch, W1, b1, W2, b2, Wl1, bl1, Wl2, bl2):
    row = edge_index[0].astype(jnp.int32)
    col = edge_index[1].astype(jnp.int32)
    row_p = jnp.pad(row, (0, EPAD - E), constant_values=0)
    col_p = jnp.pad(col, (0, EPAD - E), constant_values=NPAD - 1)

    x_p = jnp.pad(x, ((0, NPAD - N), (0, 0)))
    batch_p = jnp.pad(batch.astype(jnp.int32), (0, NPAD - N),
                      constant_values=NG).reshape(1, NPAD)

    zeros2d = jnp.zeros((NPAD, D), jnp.float32)
    zeros_c = jnp.zeros((NPAD, CW), jnp.float32)
    ones_c = jnp.ones((B, CW), jnp.float32)

    cnt = _deg_kernel(col_p, ones_c, zeros_c)
    cnt_t = cnt[:, :, 0].T

    y1, xw1 = _tc_pre(cnt_t, x_p, W1)
    acc1 = _agg_kernel(y1, row_p, col_p, zeros2d)
    y2, xw2 = _tc_mid(acc1, cnt_t, xw1, b1.reshape(1, D), W2)
    acc2 = _agg_kernel(y2, row_p, col_p, zeros2d)
    out = _tc_fin(acc2, cnt_t, xw2, b2.reshape(1, D), batch_p,
                  Wl1, bl1.reshape(1, D), Wl2, bl2.reshape(1, 16))
    return out
